# baseline (speedup 1.0000x reference)
import functools

import jax
import jax.numpy as jnp
import numpy as np
from jax import lax
from jax.experimental import pallas as pl
from jax.experimental.pallas import tpu as pltpu
from jax.experimental.pallas import tpu_sc as plsc

_B = 16384
_D = 128
_MARGIN = 1.0
_NC = 2
_NS = 16
_NW = _NC * _NS
_NSLICE = 2
_BS = _B // _NSLICE
_BPW = _BS // _NW
_C = 64
_NCH = _BPW // _C
_NPAIR = _NCH // 2
_NV = _D // 16


def _gather_transfer(h_ids, r_ids, t_ids, ent_emb, rel_emb, ent_tr, rel_tr):
    mesh = plsc.VectorSubcoreMesh(
        core_axis_name="c", subcore_axis_name="s",
        num_cores=_NC, num_subcores=_NS)
    row = jax.ShapeDtypeStruct((_BS, _D), jnp.float32)

    def body(h_ref, r_ref, t_ref, ee_ref, re_ref, et_ref, rt_ref,
             ohh, orr, ott,
             hidx, ridx, tidx, bufs, dbr, gs0, gs1, ws0, ws1):
        wid = lax.axis_index("s") * _NC + lax.axis_index("c")
        base = wid * _BPW
        pltpu.sync_copy(h_ref.at[pl.ds(base, _BPW)], hidx)
        pltpu.sync_copy(r_ref.at[pl.ds(base, _BPW)], ridx)
        pltpu.sync_copy(t_ref.at[pl.ds(base, _BPW)], tidx)
        gjobs = ((ee_ref, hidx, 0), (re_ref, ridx, 1), (ee_ref, tidx, 2),
                 (et_ref, hidx, 3), (rt_ref, ridx, 4), (et_ref, tidx, 5))
        wjobs = ((0, ohh), (1, orr), (2, ott))

        def g_desc(c, p, sem):
            off = c * _C
            return [pltpu.make_async_copy(tbl.at[idx.at[pl.ds(off, _C)]],
                                          bufs.at[p, j], sem)
                    for (tbl, idx, j) in gjobs]

        def wb_desc(c, p, sem):
            off = c * _C
            return [pltpu.make_async_copy(bufs.at[p, j],
                                          out.at[pl.ds(base + off, _C)], sem)
                    for (j, out) in wjobs]

        def transfer(p):
            def tbody(i, carry):
                for (erow, trow) in ((0, 3), (2, 5)):
                    e = bufs.at[p, erow, i]
                    et = bufs.at[p, trow, i]
                    rt = bufs.at[p, 4, i]
                    ev = [e[pl.ds(16 * v, 16)] for v in range(_NV)]
                    etv = [et[pl.ds(16 * v, 16)] for v in range(_NV)]
                    prod = ev[0] * etv[0]
                    for v in range(1, _NV):
                        prod = prod + ev[v] * etv[v]
                    d = prod
                    for step in (1, 2, 4, 8):
                        dbr[...] = d
                        idxs = lax.iota(jnp.int32, 16) ^ step
                        d = d + plsc.load_gather(dbr, [idxs])
                    for v in range(_NV):
                        e[pl.ds(16 * v, 16)] = ev[v] + d * rt[pl.ds(16 * v, 16)]
                return carry

            lax.fori_loop(0, _C, tbody, 0)

        for dsc in g_desc(0, 0, gs0):
            dsc.start()

        def pair(k, carry):
            c0 = 2 * k
            c1 = c0 + 1
            for dsc in g_desc(c0, 0, gs0):
                dsc.wait()

            @pl.when(k > 0)
            def _():
                for dsc in wb_desc(c1 - 2, 1, ws1):
                    dsc.wait()

            for dsc in g_desc(c1, 1, gs1):
                dsc.start()
            transfer(0)
            for dsc in wb_desc(c0, 0, ws0):
                dsc.start()
            for dsc in g_desc(c1, 1, gs1):
                dsc.wait()

            @pl.when(k + 1 < _NPAIR)
            def _():
                for dsc in wb_desc(c0, 0, ws0):
                    dsc.wait()
                for dsc in g_desc(c0 + 2, 0, gs0):
                    dsc.start()

            transfer(1)
            for dsc in wb_desc(c1, 1, ws1):
                dsc.start()
            return carry

        lax.fori_loop(0, _NPAIR, pair, 0)
        for dsc in wb_desc(_NCH - 2, 0, ws0):
            dsc.wait()
        for dsc in wb_desc(_NCH - 1, 1, ws1):
            dsc.wait()

    fn = pl.kernel(
        body,
        out_type=(row,) * 3,
        mesh=mesh,
        compiler_params=pltpu.CompilerParams(needs_layout_passes=False),
        scratch_types=[
            pltpu.VMEM((_BPW,), jnp.int32),
            pltpu.VMEM((_BPW,), jnp.int32),
            pltpu.VMEM((_BPW,), jnp.int32),
            pltpu.VMEM((2, 6, _C, _D), jnp.float32),
            pltpu.VMEM((16,), jnp.float32),
            pltpu.SemaphoreType.DMA,
            pltpu.SemaphoreType.DMA,
            pltpu.SemaphoreType.DMA,
            pltpu.SemaphoreType.DMA,
        ],
    )
    return fn(h_ids, r_ids, t_ids, ent_emb, rel_emb, ent_tr, rel_tr)


def _l2n(x):
    n = jnp.sqrt(jnp.sum(x * x, axis=-1, keepdims=True))
    return x / jnp.maximum(n, 1e-12)


def _score_body(hh_ref, r_ref, tt_ref, o_ref):
    hh = _l2n(hh_ref[...])
    rr = _l2n(r_ref[...])
    tt = _l2n(tt_ref[...])
    o_ref[...] = _MARGIN - jnp.sum(jnp.abs(hh + rr - tt), axis=-1)


_ROWS_PER_BLOCK = 1024
_GRID = _BS // _ROWS_PER_BLOCK


def _score(ghh, gr, gtt):
    in_spec = pl.BlockSpec((_ROWS_PER_BLOCK, _D), lambda i: (i, 0))
    return pl.pallas_call(
        _score_body,
        grid=(_GRID,),
        in_specs=[in_spec] * 3,
        out_specs=pl.BlockSpec((_ROWS_PER_BLOCK,), lambda i: (i,)),
        out_shape=jax.ShapeDtypeStruct((_BS,), jnp.float32),
    )(ghh, gr, gtt)


def kernel(sample, ent_embeddings, rel_embeddings, ent_transfer, rel_transfer):
    h_ids = sample[:, 0]
    r_ids = sample[:, 1]
    t_ids = sample[:, 2]
    scores = []
    for s in range(_NSLICE):
        lo = s * _BS
        g = _gather_transfer(
            jax.lax.dynamic_slice(h_ids, (lo,), (_BS,)),
            jax.lax.dynamic_slice(r_ids, (lo,), (_BS,)),
            jax.lax.dynamic_slice(t_ids, (lo,), (_BS,)),
            ent_embeddings, rel_embeddings, ent_transfer, rel_transfer)
        scores.append(_score(*g))
    return jnp.concatenate(scores)
